# Initial kernel scaffold; baseline (speedup 1.0000x reference)
import jax
import jax.numpy as jnp
from jax.experimental import pallas as pl


def kernel(y, train_counts, supp_counts):
    raise NotImplementedError("write your pallas kernel here")


# speedup vs baseline: 1.7341x; 1.7341x over previous
import functools

import numpy as np
import jax
import jax.numpy as jnp
from jax import lax
from jax.experimental import pallas as pl
from jax.experimental.pallas import tpu as pltpu

_ROT = ((13, 15, 26, 6), (17, 29, 16, 24))
_TINY = np.float32(np.finfo(np.float32).tiny)
_ONE_BITS = np.uint32(0x3F800000)
_P = np.float32(0.3)
_GROUP = 16


def _threefry_bits(ks0, ks1, x1):
    ks2 = ks0 ^ ks1 ^ np.uint32(0x1BD11BDA)
    ks = (ks0, ks1, ks2)
    x0 = ks0 + jnp.zeros_like(x1)
    x1 = x1 + ks1
    for i in range(5):
        for r in _ROT[i % 2]:
            x0 = x0 + x1
            x1 = (jnp.left_shift(x1, np.uint32(r))
                  | jnp.right_shift(x1, np.uint32(32 - r))) ^ x0
        x0 = x0 + ks[(i + 1) % 3]
        x1 = x1 + ks[(i + 2) % 3] + np.uint32(i + 1)
    return x0 ^ x1


def _bits_to_unit_float(bits):
    fb = jnp.right_shift(bits, np.uint32(9)) | _ONE_BITS
    return lax.bitcast_convert_type(fb, jnp.float32) - np.float32(1.0)


def _augment_body(kd_ref, lt_ref, ls_ref, y_ref, o_ref, *, L, n_classes):
    br, w = y_ref.shape
    y = y_ref[...]
    group = w // L

    lane = lax.broadcasted_iota(jnp.int32, (br, w), 1)
    seg = jnp.right_shift(lane * np.int32(41944), 8 + 15)

    yf = jnp.zeros((br, w), jnp.int32)
    for k in range(group):
        col = y[:, k * L][:, None]
        yf = jnp.where(seg == np.int32(k), col, yf)
    itv = yf == np.int32(60)

    ka0, ka1 = kd_ref[0, 0], kd_ref[0, 1]
    kb0, kb1 = kd_ref[1, 0], kd_ref[1, 1]
    kc0, kc1 = kd_ref[2, 0], kd_ref[2, 1]

    row0 = pl.program_id(0) * br
    lin = (row0 + lax.broadcasted_iota(jnp.int32, (br, w), 0)) * w + lane
    lin_u = lax.bitcast_convert_type(lin, jnp.uint32)

    u = _bits_to_unit_float(_threefry_bits(kc0, kc1, lin_u))
    noise_mask = (u < _P) & (y < np.int32(59))

    ks0 = jnp.where(itv, ka0, kb0)
    ks1 = jnp.where(itv, ka1, kb1)

    ctr0 = lax.bitcast_convert_type(lin * np.int32(n_classes), jnp.uint32)

    def cls_body(c, carry):
        best, bidx, ctr, civ = carry
        bits = _threefry_bits(ks0, ks1, ctr)
        f = _bits_to_unit_float(bits)
        uu = jnp.maximum(_TINY, f + _TINY)
        g = -jnp.log(-jnp.log(uu))
        logit = jnp.where(itv, lt_ref[c], ls_ref[c])
        val = g + logit
        upd = val > best
        best = jnp.where(upd, val, best)
        bidx = jnp.where(upd, civ, bidx)
        return best, bidx, ctr + np.uint32(1), civ + np.int32(1)

    init = (jnp.full((br, w), -np.inf, jnp.float32),
            jnp.zeros((br, w), jnp.int32),
            ctr0,
            jnp.zeros((br, w), jnp.int32))
    _, bidx, _, _ = lax.fori_loop(0, n_classes, cls_body, init)

    o_ref[...] = jnp.where(noise_mask, bidx, y)


def kernel(y, train_counts, supp_counts):
    B, L = y.shape
    n_classes = train_counts.shape[0]
    fr = B // _GROUP
    w = _GROUP * L
    br = 8 if fr % 8 == 0 else 1

    y32 = y.astype(jnp.int32).reshape(fr, w)
    kd = jax.random.key_data(jax.random.split(jax.random.key(42), 3))
    kd = kd.astype(jnp.uint32)
    lt = jnp.log(train_counts.astype(jnp.float32))
    ls = jnp.log(supp_counts.astype(jnp.float32))

    with jax.enable_x64(False):
        out = pl.pallas_call(
            functools.partial(_augment_body, L=L, n_classes=n_classes),
            grid=(fr // br,),
            in_specs=[
                pl.BlockSpec(memory_space=pltpu.SMEM),
                pl.BlockSpec(memory_space=pltpu.SMEM),
                pl.BlockSpec(memory_space=pltpu.SMEM),
                pl.BlockSpec((br, w), lambda i: (i, 0)),
            ],
            out_specs=pl.BlockSpec((br, w), lambda i: (i, 0)),
            out_shape=jax.ShapeDtypeStruct((fr, w), jnp.int32),
        )(kd, lt, ls, y32)

    return out.reshape(B, L).astype(y.dtype)
